# baseline (speedup 1.0000x reference)
import functools

import jax
import jax.numpy as jnp
from jax.experimental import pallas as pl
from jax.experimental.pallas import tpu as pltpu

TOPK_FRAC = 0.3
BLK_C = 2048
N_BATCH_BLOCKS = 4


def _loss_kernel(x_ref, t_ref, o_ref, m_ref, s_ref, g_ref, *, c_total, n_blk):
    jc = pl.program_id(1)

    @pl.when(jc == 0)
    def _init():
        m_ref[...] = jnp.full_like(m_ref, -jnp.inf)
        s_ref[...] = jnp.zeros_like(s_ref)
        g_ref[...] = jnp.full_like(g_ref, -jnp.inf)

    x = x_ref[...]
    bb, blk_c = x.shape
    tgt_rel = t_ref[...] - jc * blk_c
    col = jax.lax.broadcasted_iota(jnp.int32, (bb, blk_c), 1)

    def _update(xm):
        m_old = m_ref[...]
        m_new = jnp.maximum(m_old, jnp.max(xm, axis=1, keepdims=True))
        s_ref[...] = s_ref[...] * jnp.exp(m_old - m_new) + jnp.sum(
            jnp.exp(xm - m_new), axis=1, keepdims=True
        )
        m_ref[...] = m_new
        g_ref[...] = jnp.maximum(
            g_ref[...],
            jnp.max(jnp.where(col == tgt_rel, xm, -jnp.inf), axis=1, keepdims=True),
        )

    @pl.when(jc < n_blk - 1)
    def _main():
        _update(x)

    @pl.when(jc == n_blk - 1)
    def _tail():
        _update(jnp.where(col + jc * blk_c < c_total, x, -jnp.inf))
        o_ref[...] = m_ref[...] + jnp.log(s_ref[...]) - g_ref[...]


def _topk_mean_kernel(l_ref, o_ref, *, k):
    loss = l_ref[...]
    u = jax.lax.bitcast_convert_type(loss, jnp.uint32)
    sortable = u ^ jnp.where(
        (u >> 31) > 0, jnp.uint32(0xFFFFFFFF), jnp.uint32(0x80000000)
    )

    def body(i, th):
        cand = th | (jnp.uint32(1) << (31 - i))
        cnt = jnp.sum((sortable >= cand).astype(jnp.int32))
        return jnp.where(cnt >= k, cand, th)

    th = jax.lax.fori_loop(0, 32, body, jnp.uint32(0), unroll=True)
    gt = sortable > th
    cnt_gt = jnp.sum(gt.astype(jnp.int32))
    sum_gt = jnp.sum(jnp.where(gt, loss, 0.0))
    kth_val = jnp.max(jnp.where(sortable == th, loss, -jnp.inf))
    total = sum_gt + (k - cnt_gt).astype(jnp.float32) * kth_val
    o_ref[...] = jnp.full_like(o_ref, total / k)


def kernel(inputs, targets):
    b, c = inputs.shape
    k = max(1, int(b * TOPK_FRAC))
    n_blk = pl.cdiv(c, BLK_C)
    bb = b // N_BATCH_BLOCKS
    tgt2d = targets.reshape(b, 1)

    loss = pl.pallas_call(
        functools.partial(_loss_kernel, c_total=c, n_blk=n_blk),
        grid=(N_BATCH_BLOCKS, n_blk),
        in_specs=[
            pl.BlockSpec((bb, BLK_C), lambda i, j: (i, j)),
            pl.BlockSpec((bb, 1), lambda i, j: (i, 0)),
        ],
        out_specs=pl.BlockSpec((bb, 1), lambda i, j: (i, 0)),
        out_shape=jax.ShapeDtypeStruct((b, 1), jnp.float32),
        scratch_shapes=[
            pltpu.VMEM((bb, 1), jnp.float32),
            pltpu.VMEM((bb, 1), jnp.float32),
            pltpu.VMEM((bb, 1), jnp.float32),
        ],
        compiler_params=pltpu.CompilerParams(
            dimension_semantics=("parallel", "arbitrary")
        ),
    )(inputs, tgt2d)

    out = pl.pallas_call(
        functools.partial(_topk_mean_kernel, k=k),
        out_shape=jax.ShapeDtypeStruct((1, 1), jnp.float32),
    )(loss)
    return out.reshape(())
